# Initial kernel scaffold; baseline (speedup 1.0000x reference)
import jax
import jax.numpy as jnp
from jax.experimental import pallas as pl


def kernel(x, edge_index, W1, b1, W2, b2, Wc, bc, Wf, bf):
    raise NotImplementedError("write your pallas kernel here")


# speedup vs baseline: 9.6462x; 9.6462x over previous
import functools

import jax
import jax.numpy as jnp
from jax import lax
from jax.experimental import pallas as pl
from jax.experimental.pallas import tpu as pltpu
from jax.experimental.pallas import tpu_sc as plsc

NN = 100000
HALF = NN // 2
EE = 3200000
NC = 2
NS = 16
CH = 2048
STRIPS = CH // 128
E_PAD = 3211264
ER = E_PAD // 128
CH_ALL = 98
ACC_ROWS = 50048
SPAN = ACC_ROWS // NS
D = 16

_MESH = plsc.VectorSubcoreMesh(
    core_axis_name="c", subcore_axis_name="s", num_cores=NC, num_subcores=NS
)

_SC_PARAMS = pltpu.CompilerParams(use_tc_tiling_on_sc=False)


def _zero_rows(rows):
    z = jnp.zeros((16,), jnp.float32)

    def body(i, _):
        rows[i, :] = z
        return 0

    lax.fori_loop(0, CH, body, 0)


def _fill_ones(rows):
    o = jnp.ones((16,), jnp.float32)

    def body(i, _):
        rows[i, :] = o
        return 0

    lax.fori_loop(0, CH, body, 0)


def _zero_acc(rows, acc, s):
    base = s * SPAN
    pltpu.sync_copy(rows.at[pl.ds(0, 2048)], acc.at[pl.ds(base, 2048)])
    pltpu.sync_copy(rows.at[pl.ds(0, SPAN - 2048)],
                    acc.at[pl.ds(base + 2048, SPAN - 2048)])


_SC_SCRATCH = [
    pltpu.VMEM((STRIPS, 128), jnp.int32),
    pltpu.VMEM((STRIPS, 128), jnp.int32),
    pltpu.VMEM((CH, D), jnp.float32),
    pltpu.VMEM_SHARED((ACC_ROWS, D), jnp.float32),
    pltpu.SemaphoreType.DMA,
]


@functools.partial(
    pl.kernel,
    out_type=jax.ShapeDtypeStruct((2 * ACC_ROWS, D), jnp.float32),
    mesh=_MESH, scratch_types=_SC_SCRATCH, compiler_params=_SC_PARAMS)
def _sc_degree(lcat_hbm, out, sidx, didx, rows, acc, sem):
    c = lax.axis_index("c")
    s = lax.axis_index("s")
    _zero_rows(rows)
    _zero_acc(rows, acc, s)
    _fill_ones(rows)
    plsc.subcore_barrier()
    tile_base = c * ER + s * (CH_ALL * STRIPS)

    def chunk(it, _):
        r0 = tile_base + it * STRIPS
        pltpu.sync_copy(lcat_hbm.at[pl.ds(r0, STRIPS)], didx)
        for j in range(STRIPS):
            pltpu.sync_copy(rows.at[pl.ds(j * 128, 128)],
                            acc.at[didx.at[j]], add=True)
        return 0

    lax.fori_loop(0, CH_ALL, chunk, 0)
    plsc.subcore_barrier()
    ob = s * SPAN
    pltpu.sync_copy(acc.at[pl.ds(ob, SPAN)],
                    out.at[pl.ds(c * ACC_ROWS + ob, SPAN)])


@functools.partial(
    pl.kernel,
    out_type=jax.ShapeDtypeStruct((2 * ACC_ROWS, D), jnp.float32),
    mesh=_MESH, scratch_types=_SC_SCRATCH, compiler_params=_SC_PARAMS)
def _sc_agg1(src_hbm, lcat_hbm, table_hbm, out, sidx, didx, rows, acc, sem):
    c = lax.axis_index("c")
    s = lax.axis_index("s")
    _zero_rows(rows)
    _zero_acc(rows, acc, s)
    plsc.subcore_barrier()
    tile_base = s * (CH_ALL * STRIPS)

    def chunk(it, _):
        r0 = tile_base + it * STRIPS
        pltpu.sync_copy(src_hbm.at[pl.ds(r0, STRIPS)], sidx)
        pltpu.sync_copy(lcat_hbm.at[pl.ds(c * ER + r0, STRIPS)], didx)
        descs = [
            pltpu.async_copy(table_hbm.at[sidx.at[j]],
                             rows.at[pl.ds(j * 128, 128)], sem)
            for j in range(STRIPS)
        ]
        for d in descs:
            d.wait()
        for j in range(STRIPS):
            pltpu.sync_copy(rows.at[pl.ds(j * 128, 128)],
                            acc.at[didx.at[j]], add=True)
        return 0

    lax.fori_loop(0, CH_ALL, chunk, 0)
    plsc.subcore_barrier()
    ob = s * SPAN
    pltpu.sync_copy(acc.at[pl.ds(ob, SPAN)],
                    out.at[pl.ds(c * ACC_ROWS + ob, SPAN)])


@functools.partial(
    pl.kernel,
    out_type=jax.ShapeDtypeStruct((4 * ACC_ROWS, D), jnp.float32),
    mesh=_MESH, scratch_types=_SC_SCRATCH, compiler_params=_SC_PARAMS)
def _sc_agg2(s2cat_hbm, lcat_hbm, table_hbm, out, sidx, didx, rows, acc, sem):
    c = lax.axis_index("c")
    s = lax.axis_index("s")
    tile_base = s * (CH_ALL * STRIPS)
    for k in range(2):
        _zero_rows(rows)
        _zero_acc(rows, acc, s)
        plsc.subcore_barrier()

        def chunk(it, _):
            r0 = tile_base + it * STRIPS
            pltpu.sync_copy(s2cat_hbm.at[pl.ds(c * ER + r0, STRIPS)], sidx)
            pltpu.sync_copy(lcat_hbm.at[pl.ds(k * ER + r0, STRIPS)], didx)
            descs = [
                pltpu.async_copy(table_hbm.at[sidx.at[j]],
                                 rows.at[pl.ds(j * 128, 128)], sem)
                for j in range(STRIPS)
            ]
            for d in descs:
                d.wait()
            for j in range(STRIPS):
                pltpu.sync_copy(rows.at[pl.ds(j * 128, 128)],
                                acc.at[didx.at[j]], add=True)
            return 0

        lax.fori_loop(0, CH_ALL, chunk, 0)
        plsc.subcore_barrier()
        ob = s * SPAN
        pltpu.sync_copy(
            acc.at[pl.ds(ob, SPAN)],
            out.at[pl.ds((2 * c + k) * ACC_ROWS + ob, SPAN)])


_BLK = 5000
_GRID = NN // _BLK
_IBLK = 1568
_IGRID = ER // _IBLK


def _elu(t):
    return jnp.where(t > 0, t, jnp.exp(jnp.minimum(t, 0.0)) - 1.0)


def _row_spec(w, blk=_BLK):
    return pl.BlockSpec((blk, w), lambda i: (i, 0))


def _full_spec(shape):
    return pl.BlockSpec(shape, lambda i: tuple(0 for _ in shape))


def _tc_idx_body(src, dst, s2_o, l_o):
    k = pl.program_id(0)
    sv = src[:, :]
    dv = dst[:, :]
    s2_o[:, :] = sv + k * NN
    lo = dv - k * HALF
    ok = (lo >= 0) & (lo < HALF)
    l_o[:, :] = jnp.where(ok, lo, HALF)


def _tc_a_body(dg, x, dis_o, xs_o):
    deg = dg[:, 0:1] + 1.0
    dis = lax.rsqrt(deg)
    dis_o[:, :] = dis
    xsc = x[:, :] * dis
    xs_o[:, :] = jnp.concatenate([xsc, jnp.zeros_like(xsc)], axis=1)


def _tc_b_body(ag, xs, dis, w1, b1, w2, lo_o, hi_o):
    d = dis[:, :]
    a1 = d * (ag[:, :] + xs[:, :])
    h1 = _elu(jnp.dot(a1, w1[:, :], preferred_element_type=jnp.float32)
              + b1[:, :])
    g = jnp.dot(h1, w2[:, :], preferred_element_type=jnp.float32) * d
    lo_o[:, :] = g[:, :16]
    hi_o[:, :] = g[:, 16:]


def _tc_c1_body(alo, ahi, glo, ghi, dis, b2, h2_o):
    agg = jnp.concatenate([alo[:, :] + glo[:, :], ahi[:, :] + ghi[:, :]],
                          axis=1)
    h2_o[:, :] = _elu(dis[:, :] * agg + b2[:, :])


def _tc_c2_body(v0, v1, v2, w0, w1, w2, bc, wf, bf, out_o):
    y = (jnp.dot(v0[:, :], w0[:, :], preferred_element_type=jnp.float32)
         + jnp.dot(v1[:, :], w1[:, :], preferred_element_type=jnp.float32)
         + jnp.dot(v2[:, :], w2[:, :], preferred_element_type=jnp.float32)
         + bc[:, :])
    y = jnp.maximum(y, 0.0)
    out_o[:, :] = jnp.dot(y, wf[:, :], preferred_element_type=jnp.float32) \
        + bf[:, :]


def _halves(arr):
    return jnp.concatenate([arr[:HALF], arr[ACC_ROWS:ACC_ROWS + HALF]],
                           axis=0)


def kernel(x, edge_index, W1, b1, W2, b2, Wc, bc, Wf, bf):
    src = edge_index[0]
    dst = edge_index[1]
    pad = E_PAD - EE
    src_p = jnp.concatenate(
        [src, jnp.zeros((pad,), jnp.int32)]).reshape(ER, 128)
    dst_p = jnp.concatenate(
        [dst, jnp.full((pad,), NN, jnp.int32)]).reshape(ER, 128)

    s2cat, lcat = pl.pallas_call(
        _tc_idx_body,
        grid=(2, _IGRID),
        in_specs=[pl.BlockSpec((_IBLK, 128), lambda k, i: (i, 0)),
                  pl.BlockSpec((_IBLK, 128), lambda k, i: (i, 0))],
        out_specs=[pl.BlockSpec((_IBLK, 128), lambda k, i: (k * _IGRID + i, 0)),
                   pl.BlockSpec((_IBLK, 128), lambda k, i: (k * _IGRID + i, 0))],
        out_shape=[jax.ShapeDtypeStruct((2 * ER, 128), jnp.int32),
                   jax.ShapeDtypeStruct((2 * ER, 128), jnp.int32)],
    )(src_p, dst_p)

    dg = _halves(_sc_degree(lcat))

    dis, xs = pl.pallas_call(
        _tc_a_body,
        grid=(_GRID,),
        in_specs=[_row_spec(D), _row_spec(8)],
        out_specs=[_row_spec(1), _row_spec(D)],
        out_shape=[jax.ShapeDtypeStruct((NN, 1), jnp.float32),
                   jax.ShapeDtypeStruct((NN, D), jnp.float32)],
    )(dg, x)

    ag = _halves(_sc_agg1(src_p, lcat, xs))

    w1p = jnp.concatenate([W1, jnp.zeros((8, 64), jnp.float32)], axis=0)
    glo, ghi = pl.pallas_call(
        _tc_b_body,
        grid=(_GRID,),
        in_specs=[_row_spec(D), _row_spec(D), _row_spec(1),
                  _full_spec((16, 64)), _full_spec((1, 64)),
                  _full_spec((64, 32))],
        out_specs=[_row_spec(D), _row_spec(D)],
        out_shape=[jax.ShapeDtypeStruct((NN, D), jnp.float32),
                   jax.ShapeDtypeStruct((NN, D), jnp.float32)],
    )(ag, xs, dis, w1p, b1.reshape(1, 64), W2)

    table2 = jnp.concatenate([glo, ghi], axis=0)
    a2 = _sc_agg2(s2cat, lcat, table2)
    alo = _halves(a2[: 2 * ACC_ROWS])
    ahi = _halves(a2[2 * ACC_ROWS:])

    h2 = pl.pallas_call(
        _tc_c1_body,
        grid=(_GRID,),
        in_specs=[_row_spec(D), _row_spec(D), _row_spec(D), _row_spec(D),
                  _row_spec(1), _full_spec((1, 32))],
        out_specs=_row_spec(32),
        out_shape=jax.ShapeDtypeStruct((NN, 32), jnp.float32),
    )(alo, ahi, glo, ghi, dis, b2.reshape(1, 32))

    z1 = jnp.zeros((1, 32), jnp.float32)
    v1 = jnp.concatenate([h2[1:], z1], axis=0)
    v2 = jnp.concatenate([h2[2:], z1, z1], axis=0)

    out = pl.pallas_call(
        _tc_c2_body,
        grid=(_GRID,),
        in_specs=[_row_spec(32), _row_spec(32), _row_spec(32),
                  _full_spec((32, 16)), _full_spec((32, 16)),
                  _full_spec((32, 16)), _full_spec((1, 16)),
                  _full_spec((16, 22)), _full_spec((1, 22))],
        out_specs=_row_spec(22),
        out_shape=jax.ShapeDtypeStruct((NN, 22), jnp.float32),
    )(h2, v1, v2,
      Wc[:, :, 0].T, Wc[:, :, 1].T, Wc[:, :, 2].T, bc.reshape(1, 16),
      Wf.T, bf.reshape(1, 22))

    return out[: NN - 2]


# speedup vs baseline: 9.7595x; 1.0117x over previous
import functools

import jax
import jax.numpy as jnp
from jax import lax
from jax.experimental import pallas as pl
from jax.experimental.pallas import tpu as pltpu
from jax.experimental.pallas import tpu_sc as plsc

NN = 100000
HALF = NN // 2
EE = 3200000
NC = 2
NS = 16
CH = 2048
STRIPS = CH // 128
E_PAD = 3211264
ER = E_PAD // 128
CH_ALL = 98
ACC_ROWS = 50048
SPAN = ACC_ROWS // NS
D = 16

_MESH = plsc.VectorSubcoreMesh(
    core_axis_name="c", subcore_axis_name="s", num_cores=NC, num_subcores=NS
)

_SC_PARAMS = pltpu.CompilerParams(use_tc_tiling_on_sc=False)


def _zero_rows(rows):
    z = jnp.zeros((16,), jnp.float32)

    def body(i, _):
        rows[i, :] = z
        return 0

    lax.fori_loop(0, CH, body, 0)


def _fill_ones(rows):
    o = jnp.ones((16,), jnp.float32)

    def body(i, _):
        rows[i, :] = o
        return 0

    lax.fori_loop(0, CH, body, 0)


def _zero_acc(rows, acc, s):
    base = s * SPAN
    pltpu.sync_copy(rows.at[pl.ds(0, 2048)], acc.at[pl.ds(base, 2048)])
    pltpu.sync_copy(rows.at[pl.ds(0, SPAN - 2048)],
                    acc.at[pl.ds(base + 2048, SPAN - 2048)])


CH_PAIRS = CH_ALL // 2

_DEG_SCRATCH = [
    pltpu.VMEM((STRIPS, 128), jnp.int32),
    pltpu.VMEM((STRIPS, 128), jnp.int32),
    pltpu.VMEM((CH, D), jnp.float32),
    pltpu.VMEM_SHARED((ACC_ROWS, D), jnp.float32),
    pltpu.SemaphoreType.DMA,
    pltpu.SemaphoreType.DMA,
]

_AGG_SCRATCH = [
    pltpu.VMEM((STRIPS, 128), jnp.int32),
    pltpu.VMEM((STRIPS, 128), jnp.int32),
    pltpu.VMEM((STRIPS, 128), jnp.int32),
    pltpu.VMEM((STRIPS, 128), jnp.int32),
    pltpu.VMEM((CH, D), jnp.float32),
    pltpu.VMEM((CH, D), jnp.float32),
    pltpu.VMEM_SHARED((ACC_ROWS, D), jnp.float32),
    pltpu.SemaphoreType.DMA,
    pltpu.SemaphoreType.DMA,
    pltpu.SemaphoreType.DMA,
    pltpu.SemaphoreType.DMA,
]


def _fire_scatters(rows, didx, acc, sem):
    for j in range(STRIPS):
        pltpu.async_copy(rows.at[pl.ds(j * 128, 128)],
                         acc.at[didx.at[j]], sem, add=True)


def _drain_scatters(rows, didx, acc, sem):
    for j in range(STRIPS):
        pltpu.make_async_copy(rows.at[pl.ds(j * 128, 128)],
                              acc.at[didx.at[j]], sem).wait()


def _fire_gathers(table, sidx, rows, sem):
    for j in range(STRIPS):
        pltpu.async_copy(table.at[sidx.at[j]],
                         rows.at[pl.ds(j * 128, 128)], sem)


def _drain_gathers(table, sidx, rows, sem):
    for j in range(STRIPS):
        pltpu.make_async_copy(table.at[sidx.at[j]],
                              rows.at[pl.ds(j * 128, 128)], sem).wait()


@functools.partial(
    pl.kernel,
    out_type=jax.ShapeDtypeStruct((2 * ACC_ROWS, D), jnp.float32),
    mesh=_MESH, scratch_types=_DEG_SCRATCH, compiler_params=_SC_PARAMS)
def _sc_degree(lcat_hbm, out, didx0, didx1, rows, acc, ssem0, ssem1):
    c = lax.axis_index("c")
    s = lax.axis_index("s")
    _zero_rows(rows)
    _zero_acc(rows, acc, s)
    _fill_ones(rows)
    plsc.subcore_barrier()
    tile_base = c * ER + s * (CH_ALL * STRIPS)

    def load(i, didx):
        pltpu.sync_copy(lcat_hbm.at[pl.ds(tile_base + i * STRIPS, STRIPS)],
                        didx)

    load(0, didx0)

    def pair(t, _):
        _fire_scatters(rows, didx0, acc, ssem0)

        @pl.when(t > 0)
        def _():
            _drain_scatters(rows, didx1, acc, ssem1)
        load(2 * t + 1, didx1)
        _fire_scatters(rows, didx1, acc, ssem1)
        _drain_scatters(rows, didx0, acc, ssem0)

        @pl.when(t < CH_PAIRS - 1)
        def _():
            load(2 * t + 2, didx0)
        return 0

    lax.fori_loop(0, CH_PAIRS, pair, 0)
    _drain_scatters(rows, didx1, acc, ssem1)
    plsc.subcore_barrier()
    ob = s * SPAN
    pltpu.sync_copy(acc.at[pl.ds(ob, SPAN)],
                    out.at[pl.ds(c * ACC_ROWS + ob, SPAN)])


def _agg_pass(table_hbm, src_hbm, s_base, lcat_hbm, l_base, tile_base,
              sidx0, sidx1, didx0, didx1, rows0, rows1, acc,
              gsem0, gsem1, ssem0, ssem1):
    def load(i, sidx, didx):
        r0 = tile_base + i * STRIPS
        pltpu.sync_copy(src_hbm.at[pl.ds(s_base + r0, STRIPS)], sidx)
        pltpu.sync_copy(lcat_hbm.at[pl.ds(l_base + r0, STRIPS)], didx)

    load(0, sidx0, didx0)
    _fire_gathers(table_hbm, sidx0, rows0, gsem0)

    def pair(t, _):
        @pl.when(t > 0)
        def _():
            _drain_scatters(rows1, didx1, acc, ssem1)
        load(2 * t + 1, sidx1, didx1)
        _fire_gathers(table_hbm, sidx1, rows1, gsem1)
        _drain_gathers(table_hbm, sidx0, rows0, gsem0)
        _fire_scatters(rows0, didx0, acc, ssem0)
        _drain_scatters(rows0, didx0, acc, ssem0)

        @pl.when(t < CH_PAIRS - 1)
        def _():
            load(2 * t + 2, sidx0, didx0)
            _fire_gathers(table_hbm, sidx0, rows0, gsem0)
        _drain_gathers(table_hbm, sidx1, rows1, gsem1)
        _fire_scatters(rows1, didx1, acc, ssem1)
        return 0

    lax.fori_loop(0, CH_PAIRS, pair, 0)
    _drain_scatters(rows1, didx1, acc, ssem1)


@functools.partial(
    pl.kernel,
    out_type=jax.ShapeDtypeStruct((2 * ACC_ROWS, D), jnp.float32),
    mesh=_MESH, scratch_types=_AGG_SCRATCH, compiler_params=_SC_PARAMS)
def _sc_agg1(src_hbm, lcat_hbm, table_hbm, out, sidx0, sidx1, didx0, didx1,
             rows0, rows1, acc, gsem0, gsem1, ssem0, ssem1):
    c = lax.axis_index("c")
    s = lax.axis_index("s")
    _zero_rows(rows0)
    _zero_acc(rows0, acc, s)
    plsc.subcore_barrier()
    tile_base = s * (CH_ALL * STRIPS)
    _agg_pass(table_hbm, src_hbm, 0, lcat_hbm, c * ER, tile_base,
              sidx0, sidx1, didx0, didx1, rows0, rows1, acc,
              gsem0, gsem1, ssem0, ssem1)
    plsc.subcore_barrier()
    ob = s * SPAN
    pltpu.sync_copy(acc.at[pl.ds(ob, SPAN)],
                    out.at[pl.ds(c * ACC_ROWS + ob, SPAN)])


@functools.partial(
    pl.kernel,
    out_type=jax.ShapeDtypeStruct((4 * ACC_ROWS, D), jnp.float32),
    mesh=_MESH, scratch_types=_AGG_SCRATCH, compiler_params=_SC_PARAMS)
def _sc_agg2(s2cat_hbm, lcat_hbm, table_hbm, out, sidx0, sidx1, didx0, didx1,
             rows0, rows1, acc, gsem0, gsem1, ssem0, ssem1):
    c = lax.axis_index("c")
    s = lax.axis_index("s")
    tile_base = s * (CH_ALL * STRIPS)
    for k in range(2):
        _zero_rows(rows0)
        _zero_acc(rows0, acc, s)
        plsc.subcore_barrier()
        _agg_pass(table_hbm, s2cat_hbm, c * ER, lcat_hbm, k * ER, tile_base,
                  sidx0, sidx1, didx0, didx1, rows0, rows1, acc,
                  gsem0, gsem1, ssem0, ssem1)
        plsc.subcore_barrier()
        ob = s * SPAN
        pltpu.sync_copy(
            acc.at[pl.ds(ob, SPAN)],
            out.at[pl.ds((2 * c + k) * ACC_ROWS + ob, SPAN)])


_BLK = 5000
_GRID = NN // _BLK
_IBLK = 1568
_IGRID = ER // _IBLK


def _elu(t):
    return jnp.where(t > 0, t, jnp.exp(jnp.minimum(t, 0.0)) - 1.0)


def _row_spec(w, blk=_BLK):
    return pl.BlockSpec((blk, w), lambda i: (i, 0))


def _full_spec(shape):
    return pl.BlockSpec(shape, lambda i: tuple(0 for _ in shape))


def _tc_idx_body(src, dst, s2_o, l_o):
    k = pl.program_id(0)
    sv = src[:, :]
    dv = dst[:, :]
    s2_o[:, :] = sv + k * NN
    lo = dv - k * HALF
    ok = (lo >= 0) & (lo < HALF)
    l_o[:, :] = jnp.where(ok, lo, HALF)


def _tc_a_body(dg, x, dis_o, xs_o):
    deg = dg[:, 0:1] + 1.0
    dis = lax.rsqrt(deg)
    dis_o[:, :] = dis
    xsc = x[:, :] * dis
    xs_o[:, :] = jnp.concatenate([xsc, jnp.zeros_like(xsc)], axis=1)


def _tc_b_body(ag, xs, dis, w1, b1, w2, lo_o, hi_o):
    d = dis[:, :]
    a1 = d * (ag[:, :] + xs[:, :])
    h1 = _elu(jnp.dot(a1, w1[:, :], preferred_element_type=jnp.float32)
              + b1[:, :])
    g = jnp.dot(h1, w2[:, :], preferred_element_type=jnp.float32) * d
    lo_o[:, :] = g[:, :16]
    hi_o[:, :] = g[:, 16:]


def _tc_c1_body(alo, ahi, glo, ghi, dis, b2, h2_o):
    agg = jnp.concatenate([alo[:, :] + glo[:, :], ahi[:, :] + ghi[:, :]],
                          axis=1)
    h2_o[:, :] = _elu(dis[:, :] * agg + b2[:, :])


def _tc_c2_body(v0, v1, v2, w0, w1, w2, bc, wf, bf, out_o):
    y = (jnp.dot(v0[:, :], w0[:, :], preferred_element_type=jnp.float32)
         + jnp.dot(v1[:, :], w1[:, :], preferred_element_type=jnp.float32)
         + jnp.dot(v2[:, :], w2[:, :], preferred_element_type=jnp.float32)
         + bc[:, :])
    y = jnp.maximum(y, 0.0)
    out_o[:, :] = jnp.dot(y, wf[:, :], preferred_element_type=jnp.float32) \
        + bf[:, :]


def _halves(arr):
    return jnp.concatenate([arr[:HALF], arr[ACC_ROWS:ACC_ROWS + HALF]],
                           axis=0)


def kernel(x, edge_index, W1, b1, W2, b2, Wc, bc, Wf, bf):
    src = edge_index[0]
    dst = edge_index[1]
    pad = E_PAD - EE
    src_p = jnp.concatenate(
        [src, jnp.zeros((pad,), jnp.int32)]).reshape(ER, 128)
    dst_p = jnp.concatenate(
        [dst, jnp.full((pad,), NN, jnp.int32)]).reshape(ER, 128)

    s2cat, lcat = pl.pallas_call(
        _tc_idx_body,
        grid=(2, _IGRID),
        in_specs=[pl.BlockSpec((_IBLK, 128), lambda k, i: (i, 0)),
                  pl.BlockSpec((_IBLK, 128), lambda k, i: (i, 0))],
        out_specs=[pl.BlockSpec((_IBLK, 128), lambda k, i: (k * _IGRID + i, 0)),
                   pl.BlockSpec((_IBLK, 128), lambda k, i: (k * _IGRID + i, 0))],
        out_shape=[jax.ShapeDtypeStruct((2 * ER, 128), jnp.int32),
                   jax.ShapeDtypeStruct((2 * ER, 128), jnp.int32)],
    )(src_p, dst_p)

    dg = _halves(_sc_degree(lcat))

    dis, xs = pl.pallas_call(
        _tc_a_body,
        grid=(_GRID,),
        in_specs=[_row_spec(D), _row_spec(8)],
        out_specs=[_row_spec(1), _row_spec(D)],
        out_shape=[jax.ShapeDtypeStruct((NN, 1), jnp.float32),
                   jax.ShapeDtypeStruct((NN, D), jnp.float32)],
    )(dg, x)

    ag = _halves(_sc_agg1(src_p, lcat, xs))

    w1p = jnp.concatenate([W1, jnp.zeros((8, 64), jnp.float32)], axis=0)
    glo, ghi = pl.pallas_call(
        _tc_b_body,
        grid=(_GRID,),
        in_specs=[_row_spec(D), _row_spec(D), _row_spec(1),
                  _full_spec((16, 64)), _full_spec((1, 64)),
                  _full_spec((64, 32))],
        out_specs=[_row_spec(D), _row_spec(D)],
        out_shape=[jax.ShapeDtypeStruct((NN, D), jnp.float32),
                   jax.ShapeDtypeStruct((NN, D), jnp.float32)],
    )(ag, xs, dis, w1p, b1.reshape(1, 64), W2)

    table2 = jnp.concatenate([glo, ghi], axis=0)
    a2 = _sc_agg2(s2cat, lcat, table2)
    alo = _halves(a2[: 2 * ACC_ROWS])
    ahi = _halves(a2[2 * ACC_ROWS:])

    h2 = pl.pallas_call(
        _tc_c1_body,
        grid=(_GRID,),
        in_specs=[_row_spec(D), _row_spec(D), _row_spec(D), _row_spec(D),
                  _row_spec(1), _full_spec((1, 32))],
        out_specs=_row_spec(32),
        out_shape=jax.ShapeDtypeStruct((NN, 32), jnp.float32),
    )(alo, ahi, glo, ghi, dis, b2.reshape(1, 32))

    z1 = jnp.zeros((1, 32), jnp.float32)
    v1 = jnp.concatenate([h2[1:], z1], axis=0)
    v2 = jnp.concatenate([h2[2:], z1, z1], axis=0)

    out = pl.pallas_call(
        _tc_c2_body,
        grid=(_GRID,),
        in_specs=[_row_spec(32), _row_spec(32), _row_spec(32),
                  _full_spec((32, 16)), _full_spec((32, 16)),
                  _full_spec((32, 16)), _full_spec((1, 16)),
                  _full_spec((16, 22)), _full_spec((1, 22))],
        out_specs=_row_spec(22),
        out_shape=jax.ShapeDtypeStruct((NN, 22), jnp.float32),
    )(h2, v1, v2,
      Wc[:, :, 0].T, Wc[:, :, 1].T, Wc[:, :, 2].T, bc.reshape(1, 16),
      Wf.T, bf.reshape(1, 22))

    return out[: NN - 2]


# speedup vs baseline: 16.1317x; 1.6529x over previous
import functools

import jax
import jax.numpy as jnp
from jax import lax
from jax.experimental import pallas as pl
from jax.experimental.pallas import tpu as pltpu
from jax.experimental.pallas import tpu_sc as plsc

NN = 100000
HALF = NN // 2
EE = 3200000
NC = 2
NS = 16
E_PAD = 3211264
ER = E_PAD // 128
ACC_ROWS = 50048
SPAN = ACC_ROWS // NS
FACC_ROWS = 100096
FSPAN = FACC_ROWS // NS
D = 16

SPC_E = 8
SPC_A = 16
N_CHUNKS = 98
N_PAIRS = N_CHUNKS // 2
TSTRIDE_E = N_CHUNKS * SPC_E
TSTRIDE_A = N_CHUNKS * SPC_A

_MESH = plsc.VectorSubcoreMesh(
    core_axis_name="c", subcore_axis_name="s", num_cores=NC, num_subcores=NS
)

_SC_PARAMS = pltpu.CompilerParams(use_tc_tiling_on_sc=False)


def _fire_scatters(rows, didx, acc, sem, ns):
    for j in range(ns):
        pltpu.async_copy(rows.at[pl.ds(j * 128, 128)],
                         acc.at[didx.at[j]], sem, add=True)


def _drain_scatters(rows, didx, acc, sem, ns):
    for j in range(ns):
        pltpu.make_async_copy(rows.at[pl.ds(j * 128, 128)],
                              acc.at[didx.at[j]], sem).wait()


def _fire_gathers(table, sidx, rows, sem, ns):
    for j in range(ns):
        pltpu.async_copy(table.at[sidx.at[j]],
                         rows.at[pl.ds(j * 128, 128)], sem)


def _drain_gathers(table, sidx, rows, sem, ns):
    for j in range(ns):
        pltpu.make_async_copy(table.at[sidx.at[j]],
                              rows.at[pl.ds(j * 128, 128)], sem).wait()


_DEG_SCRATCH = [
    pltpu.VMEM((SPC_E, 128), jnp.int32),
    pltpu.VMEM((SPC_E, 128), jnp.int32),
    pltpu.VMEM((SPC_E * 128,), jnp.float32),
    pltpu.VMEM_SHARED((FACC_ROWS,), jnp.float32),
    pltpu.SemaphoreType.DMA,
    pltpu.SemaphoreType.DMA,
]


@functools.partial(
    pl.kernel,
    out_type=jax.ShapeDtypeStruct((2 * FACC_ROWS,), jnp.float32),
    mesh=_MESH, scratch_types=_DEG_SCRATCH, compiler_params=_SC_PARAMS)
def _sc_degree(dst_hbm, ones_hbm, zeros_hbm, out, didx0, didx1, ones, acc,
               ssem0, ssem1):
    c = lax.axis_index("c")
    s = lax.axis_index("s")
    pltpu.sync_copy(ones_hbm, ones)
    pltpu.sync_copy(zeros_hbm, acc.at[pl.ds(s * FSPAN, FSPAN)])
    plsc.subcore_barrier()
    tile_base = (c * NS + s) * TSTRIDE_E

    def load(i, didx):
        pltpu.sync_copy(dst_hbm.at[pl.ds(tile_base + i * SPC_E, SPC_E)],
                        didx)

    load(0, didx0)

    def pair(t, _):
        _fire_scatters(ones, didx0, acc, ssem0, SPC_E)

        @pl.when(t > 0)
        def _():
            _drain_scatters(ones, didx1, acc, ssem1, SPC_E)
        load(2 * t + 1, didx1)
        _fire_scatters(ones, didx1, acc, ssem1, SPC_E)
        _drain_scatters(ones, didx0, acc, ssem0, SPC_E)

        @pl.when(t < N_PAIRS - 1)
        def _():
            load(2 * t + 2, didx0)
        return 0

    lax.fori_loop(0, N_PAIRS, pair, 0)
    _drain_scatters(ones, didx1, acc, ssem1, SPC_E)
    plsc.subcore_barrier()
    ob = s * FSPAN
    pltpu.sync_copy(acc.at[pl.ds(ob, FSPAN)],
                    out.at[pl.ds(c * FACC_ROWS + ob, FSPAN)])


def _agg_pipeline(table_hbm, src_hbm, s_base, dst_hbm, d_base, tile_base,
                  sidx0, sidx1, didx0, didx1, rows0, rows1, acc,
                  gsem0, gsem1, ssem0, ssem1, ns):
    def load(i, sidx, didx):
        r0 = i * ns
        pltpu.sync_copy(src_hbm.at[pl.ds(s_base + tile_base + r0, ns)], sidx)
        pltpu.sync_copy(dst_hbm.at[pl.ds(d_base + tile_base + r0, ns)], didx)

    load(0, sidx0, didx0)
    _fire_gathers(table_hbm, sidx0, rows0, gsem0, ns)

    def pair(t, _):
        @pl.when(t > 0)
        def _():
            _drain_scatters(rows1, didx1, acc, ssem1, ns)
        load(2 * t + 1, sidx1, didx1)
        _fire_gathers(table_hbm, sidx1, rows1, gsem1, ns)
        _drain_gathers(table_hbm, sidx0, rows0, gsem0, ns)
        _fire_scatters(rows0, didx0, acc, ssem0, ns)
        _drain_scatters(rows0, didx0, acc, ssem0, ns)

        @pl.when(t < N_PAIRS - 1)
        def _():
            load(2 * t + 2, sidx0, didx0)
            _fire_gathers(table_hbm, sidx0, rows0, gsem0, ns)
        _drain_gathers(table_hbm, sidx1, rows1, gsem1, ns)
        _fire_scatters(rows1, didx1, acc, ssem1, ns)
        return 0

    lax.fori_loop(0, N_PAIRS, pair, 0)
    _drain_scatters(rows1, didx1, acc, ssem1, ns)


_AGG1_SCRATCH = [
    pltpu.VMEM((SPC_E, 128), jnp.int32),
    pltpu.VMEM((SPC_E, 128), jnp.int32),
    pltpu.VMEM((SPC_E, 128), jnp.int32),
    pltpu.VMEM((SPC_E, 128), jnp.int32),
    pltpu.VMEM((SPC_E * 128, 8), jnp.float32),
    pltpu.VMEM((SPC_E * 128, 8), jnp.float32),
    pltpu.VMEM_SHARED((FACC_ROWS, 8), jnp.float32),
    pltpu.SemaphoreType.DMA,
    pltpu.SemaphoreType.DMA,
    pltpu.SemaphoreType.DMA,
    pltpu.SemaphoreType.DMA,
]


@functools.partial(
    pl.kernel,
    out_type=jax.ShapeDtypeStruct((2 * FACC_ROWS, 8), jnp.float32),
    mesh=_MESH, scratch_types=_AGG1_SCRATCH, compiler_params=_SC_PARAMS)
def _sc_agg1(src_hbm, dst_hbm, table_hbm, zeros_hbm, out,
             sidx0, sidx1, didx0, didx1, rows0, rows1, acc,
             gsem0, gsem1, ssem0, ssem1):
    c = lax.axis_index("c")
    s = lax.axis_index("s")
    pltpu.sync_copy(zeros_hbm, acc.at[pl.ds(s * FSPAN, FSPAN)])
    plsc.subcore_barrier()
    tile_base = (c * NS + s) * TSTRIDE_E
    _agg_pipeline(table_hbm, src_hbm, 0, dst_hbm, 0, tile_base,
                  sidx0, sidx1, didx0, didx1, rows0, rows1, acc,
                  gsem0, gsem1, ssem0, ssem1, SPC_E)
    plsc.subcore_barrier()
    ob = s * FSPAN
    pltpu.sync_copy(acc.at[pl.ds(ob, FSPAN)],
                    out.at[pl.ds(c * FACC_ROWS + ob, FSPAN)])


_AGG2_SCRATCH = [
    pltpu.VMEM((SPC_A, 128), jnp.int32),
    pltpu.VMEM((SPC_A, 128), jnp.int32),
    pltpu.VMEM((SPC_A, 128), jnp.int32),
    pltpu.VMEM((SPC_A, 128), jnp.int32),
    pltpu.VMEM((SPC_A * 128, D), jnp.float32),
    pltpu.VMEM((SPC_A * 128, D), jnp.float32),
    pltpu.VMEM_SHARED((ACC_ROWS, D), jnp.float32),
    pltpu.SemaphoreType.DMA,
    pltpu.SemaphoreType.DMA,
    pltpu.SemaphoreType.DMA,
    pltpu.SemaphoreType.DMA,
]


@functools.partial(
    pl.kernel,
    out_type=jax.ShapeDtypeStruct((4 * ACC_ROWS, D), jnp.float32),
    mesh=_MESH, scratch_types=_AGG2_SCRATCH, compiler_params=_SC_PARAMS)
def _sc_agg2(s2cat_hbm, lcat_hbm, table_hbm, zeros_hbm, out,
             sidx0, sidx1, didx0, didx1, rows0, rows1, acc,
             gsem0, gsem1, ssem0, ssem1):
    c = lax.axis_index("c")
    s = lax.axis_index("s")
    tile_base = s * TSTRIDE_A
    for k in range(2):
        pltpu.sync_copy(zeros_hbm, acc.at[pl.ds(s * SPAN, SPAN)])
        plsc.subcore_barrier()
        _agg_pipeline(table_hbm, s2cat_hbm, c * ER, lcat_hbm, k * ER,
                      tile_base, sidx0, sidx1, didx0, didx1, rows0, rows1,
                      acc, gsem0, gsem1, ssem0, ssem1, SPC_A)
        plsc.subcore_barrier()
        ob = s * SPAN
        pltpu.sync_copy(
            acc.at[pl.ds(ob, SPAN)],
            out.at[pl.ds((2 * c + k) * ACC_ROWS + ob, SPAN)])


_BLK = 5000
_GRID = NN // _BLK
_IBLK = 1568
_IGRID = ER // _IBLK


def _elu(t):
    return jnp.where(t > 0, t, jnp.exp(jnp.minimum(t, 0.0)) - 1.0)


def _row_spec(w, blk=_BLK):
    return pl.BlockSpec((blk, w), lambda i: (i, 0))


def _full_spec(shape):
    return pl.BlockSpec(shape, lambda i: tuple(0 for _ in shape))


def _tc_idx_body(src, dst, s2_o, l_o):
    k = pl.program_id(0)
    sv = src[:, :]
    dv = dst[:, :]
    s2_o[:, :] = sv + k * NN
    lo = dv - k * HALF
    ok = (lo >= 0) & (lo < HALF)
    l_o[:, :] = jnp.where(ok, lo, HALF)


def _tc_a_body(d0, d1, x, dis_o, xs_o):
    deg = d0[:, :] + d1[:, :] + 1.0
    dis = lax.rsqrt(deg)
    dis_o[:, :] = dis
    xs_o[:, :] = x[:, :] * dis


def _tc_b_body(ag0, ag1, xs, dis, w1, b1, w2, lo_o, hi_o):
    d = dis[:, :]
    a1 = d * (ag0[:, :] + ag1[:, :] + xs[:, :])
    h1 = _elu(jnp.dot(a1, w1[:, :], preferred_element_type=jnp.float32)
              + b1[:, :])
    g = jnp.dot(h1, w2[:, :], preferred_element_type=jnp.float32) * d
    lo_o[:, :] = g[:, :16]
    hi_o[:, :] = g[:, 16:]


def _tc_c1_body(alo, ahi, glo, ghi, dis, b2, h2_o):
    agg = jnp.concatenate([alo[:, :] + glo[:, :], ahi[:, :] + ghi[:, :]],
                          axis=1)
    h2_o[:, :] = _elu(dis[:, :] * agg + b2[:, :])


def _tc_c2_body(v0, v1, v2, w0, w1, w2, bc, wf, bf, out_o):
    y = (jnp.dot(v0[:, :], w0[:, :], preferred_element_type=jnp.float32)
         + jnp.dot(v1[:, :], w1[:, :], preferred_element_type=jnp.float32)
         + jnp.dot(v2[:, :], w2[:, :], preferred_element_type=jnp.float32)
         + bc[:, :])
    y = jnp.maximum(y, 0.0)
    out_o[:, :] = jnp.dot(y, wf[:, :], preferred_element_type=jnp.float32) \
        + bf[:, :]


def _halves(arr):
    return jnp.concatenate([arr[:HALF], arr[ACC_ROWS:ACC_ROWS + HALF]],
                           axis=0)


def kernel(x, edge_index, W1, b1, W2, b2, Wc, bc, Wf, bf):
    src = edge_index[0]
    dst = edge_index[1]
    pad = E_PAD - EE
    src_p = jnp.concatenate(
        [src, jnp.zeros((pad,), jnp.int32)]).reshape(ER, 128)
    dst_p = jnp.concatenate(
        [dst, jnp.full((pad,), NN, jnp.int32)]).reshape(ER, 128)

    s2cat, lcat = pl.pallas_call(
        _tc_idx_body,
        grid=(2, _IGRID),
        in_specs=[pl.BlockSpec((_IBLK, 128), lambda k, i: (i, 0)),
                  pl.BlockSpec((_IBLK, 128), lambda k, i: (i, 0))],
        out_specs=[pl.BlockSpec((_IBLK, 128), lambda k, i: (k * _IGRID + i, 0)),
                   pl.BlockSpec((_IBLK, 128), lambda k, i: (k * _IGRID + i, 0))],
        out_shape=[jax.ShapeDtypeStruct((2 * ER, 128), jnp.int32),
                   jax.ShapeDtypeStruct((2 * ER, 128), jnp.int32)],
    )(src_p, dst_p)

    ones_e = jnp.ones((SPC_E * 128,), jnp.float32)
    zeros_1 = jnp.zeros((FSPAN,), jnp.float32)
    zeros_8 = jnp.zeros((FSPAN, 8), jnp.float32)
    zeros_16 = jnp.zeros((SPAN, D), jnp.float32)

    dgout = _sc_degree(dst_p, ones_e, zeros_1)
    d0 = dgout[:NN].reshape(NN, 1)
    d1 = dgout[FACC_ROWS:FACC_ROWS + NN].reshape(NN, 1)

    dis, xs = pl.pallas_call(
        _tc_a_body,
        grid=(_GRID,),
        in_specs=[_row_spec(1), _row_spec(1), _row_spec(8)],
        out_specs=[_row_spec(1), _row_spec(8)],
        out_shape=[jax.ShapeDtypeStruct((NN, 1), jnp.float32),
                   jax.ShapeDtypeStruct((NN, 8), jnp.float32)],
    )(d0, d1, x)

    agout = _sc_agg1(src_p, dst_p, xs, zeros_8)
    ag0 = agout[:NN]
    ag1 = agout[FACC_ROWS:FACC_ROWS + NN]

    glo, ghi = pl.pallas_call(
        _tc_b_body,
        grid=(_GRID,),
        in_specs=[_row_spec(8), _row_spec(8), _row_spec(8), _row_spec(1),
                  _full_spec((8, 64)), _full_spec((1, 64)),
                  _full_spec((64, 32))],
        out_specs=[_row_spec(D), _row_spec(D)],
        out_shape=[jax.ShapeDtypeStruct((NN, D), jnp.float32),
                   jax.ShapeDtypeStruct((NN, D), jnp.float32)],
    )(ag0, ag1, xs, dis, W1, b1.reshape(1, 64), W2)

    table2 = jnp.concatenate([glo, ghi], axis=0)
    a2 = _sc_agg2(s2cat, lcat, table2, zeros_16)
    alo = _halves(a2[: 2 * ACC_ROWS])
    ahi = _halves(a2[2 * ACC_ROWS:])

    h2 = pl.pallas_call(
        _tc_c1_body,
        grid=(_GRID,),
        in_specs=[_row_spec(D), _row_spec(D), _row_spec(D), _row_spec(D),
                  _row_spec(1), _full_spec((1, 32))],
        out_specs=_row_spec(32),
        out_shape=jax.ShapeDtypeStruct((NN, 32), jnp.float32),
    )(alo, ahi, glo, ghi, dis, b2.reshape(1, 32))

    z1 = jnp.zeros((1, 32), jnp.float32)
    v1 = jnp.concatenate([h2[1:], z1], axis=0)
    v2 = jnp.concatenate([h2[2:], z1, z1], axis=0)

    out = pl.pallas_call(
        _tc_c2_body,
        grid=(_GRID,),
        in_specs=[_row_spec(32), _row_spec(32), _row_spec(32),
                  _full_spec((32, 16)), _full_spec((32, 16)),
                  _full_spec((32, 16)), _full_spec((1, 16)),
                  _full_spec((16, 22)), _full_spec((1, 22))],
        out_specs=_row_spec(22),
        out_shape=jax.ShapeDtypeStruct((NN, 22), jnp.float32),
    )(h2, v1, v2,
      Wc[:, :, 0].T, Wc[:, :, 1].T, Wc[:, :, 2].T, bc.reshape(1, 16),
      Wf.T, bf.reshape(1, 22))

    return out[: NN - 2]


# speedup vs baseline: 37.6265x; 2.3325x over previous
import functools

import jax
import jax.numpy as jnp
from jax import lax
from jax.experimental import pallas as pl
from jax.experimental.pallas import tpu as pltpu
from jax.experimental.pallas import tpu_sc as plsc

NN = 100000
HALF = NN // 2
EE = 3200000
NC = 2
NS = 16
E_PAD = 3211264
ER = E_PAD // 128
ACC_ROWS = 50048
SPAN = ACC_ROWS // NS
FACC_ROWS = 100096
FSPAN = FACC_ROWS // NS
D = 16

SPC_E = 8
SPC_A = 16
N_CHUNKS = 98
N_PAIRS = N_CHUNKS // 2
TSTRIDE_E = N_CHUNKS * SPC_E
TSTRIDE_A = N_CHUNKS * SPC_A

_MESH = plsc.VectorSubcoreMesh(
    core_axis_name="c", subcore_axis_name="s", num_cores=NC, num_subcores=NS
)

_SC_PARAMS = pltpu.CompilerParams(use_tc_tiling_on_sc=False)


def _fire_scatters(rows, didx, acc, sem, ns):
    for j in range(ns):
        pltpu.async_copy(rows.at[pl.ds(j * 128, 128)],
                         acc.at[didx.at[j]], sem, add=True)


def _drain_scatters(rows, didx, acc, sem, ns):
    for j in range(ns):
        pltpu.make_async_copy(rows.at[pl.ds(j * 128, 128)],
                              acc.at[didx.at[j]], sem).wait()


def _fire_gathers(table, sidx, rows, sem, ns):
    for j in range(ns):
        pltpu.async_copy(table.at[sidx.at[j]],
                         rows.at[pl.ds(j * 128, 128)], sem)


def _drain_gathers(table, sidx, rows, sem, ns):
    for j in range(ns):
        pltpu.make_async_copy(table.at[sidx.at[j]],
                              rows.at[pl.ds(j * 128, 128)], sem).wait()


_DEG_SCRATCH = [
    pltpu.VMEM((SPC_E, 128), jnp.int32),
    pltpu.VMEM((SPC_E, 128), jnp.int32),
    pltpu.VMEM((SPC_E * 128,), jnp.float32),
    pltpu.VMEM_SHARED((FACC_ROWS,), jnp.float32),
    pltpu.SemaphoreType.DMA,
    pltpu.SemaphoreType.DMA,
]


@functools.partial(
    pl.kernel,
    out_type=jax.ShapeDtypeStruct((2 * FACC_ROWS,), jnp.float32),
    mesh=_MESH, scratch_types=_DEG_SCRATCH, compiler_params=_SC_PARAMS)
def _sc_degree(dst_hbm, ones_hbm, zeros_hbm, out, didx0, didx1, ones, acc,
               ssem0, ssem1):
    c = lax.axis_index("c")
    s = lax.axis_index("s")
    pltpu.sync_copy(ones_hbm, ones)
    pltpu.sync_copy(zeros_hbm, acc.at[pl.ds(s * FSPAN, FSPAN)])
    plsc.subcore_barrier()
    tile_base = (c * NS + s) * TSTRIDE_E

    def load(i, didx):
        pltpu.sync_copy(dst_hbm.at[pl.ds(tile_base + i * SPC_E, SPC_E)],
                        didx)

    load(0, didx0)

    def pair(t, _):
        _fire_scatters(ones, didx0, acc, ssem0, SPC_E)

        @pl.when(t > 0)
        def _():
            _drain_scatters(ones, didx1, acc, ssem1, SPC_E)
        load(2 * t + 1, didx1)
        _fire_scatters(ones, didx1, acc, ssem1, SPC_E)
        _drain_scatters(ones, didx0, acc, ssem0, SPC_E)

        @pl.when(t < N_PAIRS - 1)
        def _():
            load(2 * t + 2, didx0)
        return 0

    lax.fori_loop(0, N_PAIRS, pair, 0)
    _drain_scatters(ones, didx1, acc, ssem1, SPC_E)
    plsc.subcore_barrier()
    ob = s * FSPAN
    pltpu.sync_copy(acc.at[pl.ds(ob, FSPAN)],
                    out.at[pl.ds(c * FACC_ROWS + ob, FSPAN)])


def _agg_pipeline(table_hbm, src_hbm, s_base, dst_hbm, d_base, tile_base,
                  sidx0, sidx1, didx0, didx1, rows0, rows1, acc,
                  gsem0, gsem1, ssem0, ssem1, ns):
    def load(i, sidx, didx):
        r0 = i * ns
        pltpu.sync_copy(src_hbm.at[pl.ds(s_base + tile_base + r0, ns)], sidx)
        pltpu.sync_copy(dst_hbm.at[pl.ds(d_base + tile_base + r0, ns)], didx)

    load(0, sidx0, didx0)
    _fire_gathers(table_hbm, sidx0, rows0, gsem0, ns)

    def pair(t, _):
        @pl.when(t > 0)
        def _():
            _drain_scatters(rows1, didx1, acc, ssem1, ns)
        load(2 * t + 1, sidx1, didx1)
        _fire_gathers(table_hbm, sidx1, rows1, gsem1, ns)
        _drain_gathers(table_hbm, sidx0, rows0, gsem0, ns)
        _fire_scatters(rows0, didx0, acc, ssem0, ns)
        _drain_scatters(rows0, didx0, acc, ssem0, ns)

        @pl.when(t < N_PAIRS - 1)
        def _():
            load(2 * t + 2, sidx0, didx0)
            _fire_gathers(table_hbm, sidx0, rows0, gsem0, ns)
        _drain_gathers(table_hbm, sidx1, rows1, gsem1, ns)
        _fire_scatters(rows1, didx1, acc, ssem1, ns)
        return 0

    lax.fori_loop(0, N_PAIRS, pair, 0)
    _drain_scatters(rows1, didx1, acc, ssem1, ns)


_AGG1_SCRATCH = [
    pltpu.VMEM((SPC_E, 128), jnp.int32),
    pltpu.VMEM((SPC_E, 128), jnp.int32),
    pltpu.VMEM((SPC_E, 128), jnp.int32),
    pltpu.VMEM((SPC_E, 128), jnp.int32),
    pltpu.VMEM((SPC_E * 128, 8), jnp.float32),
    pltpu.VMEM((SPC_E * 128, 8), jnp.float32),
    pltpu.VMEM_SHARED((FACC_ROWS, 8), jnp.float32),
    pltpu.SemaphoreType.DMA,
    pltpu.SemaphoreType.DMA,
    pltpu.SemaphoreType.DMA,
    pltpu.SemaphoreType.DMA,
]


@functools.partial(
    pl.kernel,
    out_type=jax.ShapeDtypeStruct((2 * FACC_ROWS, 8), jnp.float32),
    mesh=_MESH, scratch_types=_AGG1_SCRATCH, compiler_params=_SC_PARAMS)
def _sc_agg1(src_hbm, dst_hbm, table_hbm, zeros_hbm, out,
             sidx0, sidx1, didx0, didx1, rows0, rows1, acc,
             gsem0, gsem1, ssem0, ssem1):
    c = lax.axis_index("c")
    s = lax.axis_index("s")
    pltpu.sync_copy(zeros_hbm, acc.at[pl.ds(s * FSPAN, FSPAN)])
    plsc.subcore_barrier()
    tile_base = (c * NS + s) * TSTRIDE_E
    _agg_pipeline(table_hbm, src_hbm, 0, dst_hbm, 0, tile_base,
                  sidx0, sidx1, didx0, didx1, rows0, rows1, acc,
                  gsem0, gsem1, ssem0, ssem1, SPC_E)
    plsc.subcore_barrier()
    ob = s * FSPAN
    pltpu.sync_copy(acc.at[pl.ds(ob, FSPAN)],
                    out.at[pl.ds(c * FACC_ROWS + ob, FSPAN)])


_AGG2_SCRATCH = [
    pltpu.VMEM((SPC_A, 128), jnp.int32),
    pltpu.VMEM((SPC_A, 128), jnp.int32),
    pltpu.VMEM((SPC_A, 128), jnp.int32),
    pltpu.VMEM((SPC_A, 128), jnp.int32),
    pltpu.VMEM((SPC_A * 128, D), jnp.float32),
    pltpu.VMEM((SPC_A * 128, D), jnp.float32),
    pltpu.VMEM_SHARED((ACC_ROWS, D), jnp.float32),
    pltpu.SemaphoreType.DMA,
    pltpu.SemaphoreType.DMA,
    pltpu.SemaphoreType.DMA,
    pltpu.SemaphoreType.DMA,
]


@functools.partial(
    pl.kernel,
    out_type=jax.ShapeDtypeStruct((4 * ACC_ROWS, D), jnp.float32),
    mesh=_MESH, scratch_types=_AGG2_SCRATCH, compiler_params=_SC_PARAMS)
def _sc_agg2(s2cat_hbm, lcat_hbm, table_hbm, zeros_hbm, out,
             sidx0, sidx1, didx0, didx1, rows0, rows1, acc,
             gsem0, gsem1, ssem0, ssem1):
    c = lax.axis_index("c")
    s = lax.axis_index("s")
    tile_base = s * TSTRIDE_A
    for k in range(2):
        pltpu.sync_copy(zeros_hbm, acc.at[pl.ds(s * SPAN, SPAN)])
        plsc.subcore_barrier()
        _agg_pipeline(table_hbm, s2cat_hbm, c * ER, lcat_hbm, k * ER,
                      tile_base, sidx0, sidx1, didx0, didx1, rows0, rows1,
                      acc, gsem0, gsem1, ssem0, ssem1, SPC_A)
        plsc.subcore_barrier()
        ob = s * SPAN
        pltpu.sync_copy(
            acc.at[pl.ds(ob, SPAN)],
            out.at[pl.ds((2 * c + k) * ACC_ROWS + ob, SPAN)])


_BLK = 5000
_GRID = NN // _BLK
_IBLK = 1568
_IGRID = ER // _IBLK


def _elu(t):
    return jnp.where(t > 0, t, jnp.exp(jnp.minimum(t, 0.0)) - 1.0)


def _row_spec(w, blk=_BLK):
    return pl.BlockSpec((blk, w), lambda i: (i, 0))


def _full_spec(shape):
    return pl.BlockSpec(shape, lambda i: tuple(0 for _ in shape))


def _tc_idx_body(src, dst, s2_o, l_o):
    k = pl.program_id(0)
    sv = src[:, :]
    dv = dst[:, :]
    s2_o[:, :] = sv + k * NN
    lo = dv - k * HALF
    ok = (lo >= 0) & (lo < HALF)
    dummy = HALF + (dv & 31)
    l_o[:, :] = jnp.where(ok, lo, dummy)


def _tc_a_body(d0, d1, x, dis_o, xs_o):
    deg = d0[:, :] + d1[:, :] + 1.0
    dis = lax.rsqrt(deg)
    dis_o[:, :] = dis
    xs_o[:, :] = x[:, :] * dis


def _tc_b_body(ag0, ag1, xs, dis, w1, b1, w2, lo_o, hi_o):
    d = dis[:, :]
    a1 = d * (ag0[:, :] + ag1[:, :] + xs[:, :])
    h1 = _elu(jnp.dot(a1, w1[:, :], preferred_element_type=jnp.float32)
              + b1[:, :])
    g = jnp.dot(h1, w2[:, :], preferred_element_type=jnp.float32) * d
    lo_o[:, :] = g[:, :16]
    hi_o[:, :] = g[:, 16:]


def _tc_c1_body(alo, ahi, glo, ghi, dis, b2, h2_o):
    agg = jnp.concatenate([alo[:, :] + glo[:, :], ahi[:, :] + ghi[:, :]],
                          axis=1)
    h2_o[:, :] = _elu(dis[:, :] * agg + b2[:, :])


def _tc_c2_body(v0, v1, v2, w0, w1, w2, bc, wf, bf, out_o):
    y = (jnp.dot(v0[:, :], w0[:, :], preferred_element_type=jnp.float32)
         + jnp.dot(v1[:, :], w1[:, :], preferred_element_type=jnp.float32)
         + jnp.dot(v2[:, :], w2[:, :], preferred_element_type=jnp.float32)
         + bc[:, :])
    y = jnp.maximum(y, 0.0)
    out_o[:, :] = jnp.dot(y, wf[:, :], preferred_element_type=jnp.float32) \
        + bf[:, :]


def _halves(arr):
    return jnp.concatenate([arr[:HALF], arr[ACC_ROWS:ACC_ROWS + HALF]],
                           axis=0)


def kernel(x, edge_index, W1, b1, W2, b2, Wc, bc, Wf, bf):
    src = edge_index[0]
    dst = edge_index[1]
    pad = E_PAD - EE
    src_p = jnp.concatenate(
        [src, jnp.zeros((pad,), jnp.int32)]).reshape(ER, 128)
    dst_p = jnp.concatenate(
        [dst, jnp.full((pad,), NN, jnp.int32)]).reshape(ER, 128)

    s2cat, lcat = pl.pallas_call(
        _tc_idx_body,
        grid=(2, _IGRID),
        in_specs=[pl.BlockSpec((_IBLK, 128), lambda k, i: (i, 0)),
                  pl.BlockSpec((_IBLK, 128), lambda k, i: (i, 0))],
        out_specs=[pl.BlockSpec((_IBLK, 128), lambda k, i: (k * _IGRID + i, 0)),
                   pl.BlockSpec((_IBLK, 128), lambda k, i: (k * _IGRID + i, 0))],
        out_shape=[jax.ShapeDtypeStruct((2 * ER, 128), jnp.int32),
                   jax.ShapeDtypeStruct((2 * ER, 128), jnp.int32)],
    )(src_p, dst_p)

    ones_e = jnp.ones((SPC_E * 128,), jnp.float32)
    zeros_1 = jnp.zeros((FSPAN,), jnp.float32)
    zeros_8 = jnp.zeros((FSPAN, 8), jnp.float32)
    zeros_16 = jnp.zeros((SPAN, D), jnp.float32)

    dgout = _sc_degree(dst_p, ones_e, zeros_1)
    d0 = dgout[:NN].reshape(NN, 1)
    d1 = dgout[FACC_ROWS:FACC_ROWS + NN].reshape(NN, 1)

    dis, xs = pl.pallas_call(
        _tc_a_body,
        grid=(_GRID,),
        in_specs=[_row_spec(1), _row_spec(1), _row_spec(8)],
        out_specs=[_row_spec(1), _row_spec(8)],
        out_shape=[jax.ShapeDtypeStruct((NN, 1), jnp.float32),
                   jax.ShapeDtypeStruct((NN, 8), jnp.float32)],
    )(d0, d1, x)

    agout = _sc_agg1(src_p, dst_p, xs, zeros_8)
    ag0 = agout[:NN]
    ag1 = agout[FACC_ROWS:FACC_ROWS + NN]

    glo, ghi = pl.pallas_call(
        _tc_b_body,
        grid=(_GRID,),
        in_specs=[_row_spec(8), _row_spec(8), _row_spec(8), _row_spec(1),
                  _full_spec((8, 64)), _full_spec((1, 64)),
                  _full_spec((64, 32))],
        out_specs=[_row_spec(D), _row_spec(D)],
        out_shape=[jax.ShapeDtypeStruct((NN, D), jnp.float32),
                   jax.ShapeDtypeStruct((NN, D), jnp.float32)],
    )(ag0, ag1, xs, dis, W1, b1.reshape(1, 64), W2)

    table2 = jnp.concatenate([glo, ghi], axis=0)
    a2 = _sc_agg2(s2cat, lcat, table2, zeros_16)
    alo = _halves(a2[: 2 * ACC_ROWS])
    ahi = _halves(a2[2 * ACC_ROWS:])

    h2 = pl.pallas_call(
        _tc_c1_body,
        grid=(_GRID,),
        in_specs=[_row_spec(D), _row_spec(D), _row_spec(D), _row_spec(D),
                  _row_spec(1), _full_spec((1, 32))],
        out_specs=_row_spec(32),
        out_shape=jax.ShapeDtypeStruct((NN, 32), jnp.float32),
    )(alo, ahi, glo, ghi, dis, b2.reshape(1, 32))

    z1 = jnp.zeros((1, 32), jnp.float32)
    v1 = jnp.concatenate([h2[1:], z1], axis=0)
    v2 = jnp.concatenate([h2[2:], z1, z1], axis=0)

    out = pl.pallas_call(
        _tc_c2_body,
        grid=(_GRID,),
        in_specs=[_row_spec(32), _row_spec(32), _row_spec(32),
                  _full_spec((32, 16)), _full_spec((32, 16)),
                  _full_spec((32, 16)), _full_spec((1, 16)),
                  _full_spec((16, 22)), _full_spec((1, 22))],
        out_specs=_row_spec(22),
        out_shape=jax.ShapeDtypeStruct((NN, 22), jnp.float32),
    )(h2, v1, v2,
      Wc[:, :, 0].T, Wc[:, :, 1].T, Wc[:, :, 2].T, bc.reshape(1, 16),
      Wf.T, bf.reshape(1, 22))

    return out[: NN - 2]


# speedup vs baseline: 41.5410x; 1.1040x over previous
import functools

import jax
import jax.numpy as jnp
from jax import lax
from jax.experimental import pallas as pl
from jax.experimental.pallas import tpu as pltpu
from jax.experimental.pallas import tpu_sc as plsc

NN = 100000
HALF = NN // 2
EE = 3200000
NC = 2
NS = 16
E_PAD = 3211264
ER = E_PAD // 128
ACC_ROWS = 50048
SPAN = ACC_ROWS // NS
FACC_ROWS = 100096
FSPAN = FACC_ROWS // NS
D = 16

SPC_E = 8
SPC_A = 16
N_CHUNKS = 98
N_PAIRS = N_CHUNKS // 2
TSTRIDE_E = N_CHUNKS * SPC_E
TSTRIDE_A = N_CHUNKS * SPC_A

_MESH = plsc.VectorSubcoreMesh(
    core_axis_name="c", subcore_axis_name="s", num_cores=NC, num_subcores=NS
)

_SC_PARAMS = pltpu.CompilerParams(use_tc_tiling_on_sc=False)


def _fire_scatters(rows, didx, acc, sem, ns):
    for j in range(ns):
        pltpu.async_copy(rows.at[pl.ds(j * 128, 128)],
                         acc.at[didx.at[j]], sem, add=True)


def _drain_scatters(rows, didx, acc, sem, ns):
    for j in range(ns):
        pltpu.make_async_copy(rows.at[pl.ds(j * 128, 128)],
                              acc.at[didx.at[j]], sem).wait()


def _fire_gathers(table, sidx, rows, sem, ns):
    for j in range(ns):
        pltpu.async_copy(table.at[sidx.at[j]],
                         rows.at[pl.ds(j * 128, 128)], sem)


def _drain_gathers(table, sidx, rows, sem, ns):
    for j in range(ns):
        pltpu.make_async_copy(table.at[sidx.at[j]],
                              rows.at[pl.ds(j * 128, 128)], sem).wait()


_DEG_SCRATCH = [
    pltpu.VMEM((SPC_E, 128), jnp.int32),
    pltpu.VMEM((SPC_E, 128), jnp.int32),
    pltpu.VMEM((SPC_E * 128,), jnp.float32),
    pltpu.VMEM_SHARED((FACC_ROWS,), jnp.float32),
    pltpu.SemaphoreType.DMA,
    pltpu.SemaphoreType.DMA,
]


@functools.partial(
    pl.kernel,
    out_type=jax.ShapeDtypeStruct((2 * FACC_ROWS,), jnp.float32),
    mesh=_MESH, scratch_types=_DEG_SCRATCH, compiler_params=_SC_PARAMS)
def _sc_degree(dst_hbm, ones_hbm, zeros_hbm, out, didx0, didx1, ones, acc,
               ssem0, ssem1):
    c = lax.axis_index("c")
    s = lax.axis_index("s")
    pltpu.sync_copy(ones_hbm, ones)
    pltpu.sync_copy(zeros_hbm, acc.at[pl.ds(s * FSPAN, FSPAN)])
    plsc.subcore_barrier()
    tile_base = (c * NS + s) * TSTRIDE_E

    def load(i, didx):
        pltpu.sync_copy(dst_hbm.at[pl.ds(tile_base + i * SPC_E, SPC_E)],
                        didx)

    load(0, didx0)

    def pair(t, _):
        _fire_scatters(ones, didx0, acc, ssem0, SPC_E)

        @pl.when(t > 0)
        def _():
            _drain_scatters(ones, didx1, acc, ssem1, SPC_E)
        load(2 * t + 1, didx1)
        _fire_scatters(ones, didx1, acc, ssem1, SPC_E)
        _drain_scatters(ones, didx0, acc, ssem0, SPC_E)

        @pl.when(t < N_PAIRS - 1)
        def _():
            load(2 * t + 2, didx0)
        return 0

    lax.fori_loop(0, N_PAIRS, pair, 0)
    _drain_scatters(ones, didx1, acc, ssem1, SPC_E)
    plsc.subcore_barrier()
    ob = s * FSPAN
    pltpu.sync_copy(acc.at[pl.ds(ob, FSPAN)],
                    out.at[pl.ds(c * FACC_ROWS + ob, FSPAN)])


def _agg_pipeline(table_hbm, src_hbm, s_base, dst_hbm, d_base, tile_base,
                  sidx0, sidx1, didx0, didx1, rows0, rows1, acc,
                  gsem0, gsem1, ssem0, ssem1, ns):
    def load(i, sidx, didx):
        r0 = i * ns
        pltpu.sync_copy(src_hbm.at[pl.ds(s_base + tile_base + r0, ns)], sidx)
        pltpu.sync_copy(dst_hbm.at[pl.ds(d_base + tile_base + r0, ns)], didx)

    load(0, sidx0, didx0)
    _fire_gathers(table_hbm, sidx0, rows0, gsem0, ns)

    def pair(t, _):
        @pl.when(t > 0)
        def _():
            _drain_scatters(rows1, didx1, acc, ssem1, ns)
        load(2 * t + 1, sidx1, didx1)
        _fire_gathers(table_hbm, sidx1, rows1, gsem1, ns)
        _drain_gathers(table_hbm, sidx0, rows0, gsem0, ns)
        _fire_scatters(rows0, didx0, acc, ssem0, ns)
        _drain_scatters(rows0, didx0, acc, ssem0, ns)

        @pl.when(t < N_PAIRS - 1)
        def _():
            load(2 * t + 2, sidx0, didx0)
            _fire_gathers(table_hbm, sidx0, rows0, gsem0, ns)
        _drain_gathers(table_hbm, sidx1, rows1, gsem1, ns)
        _fire_scatters(rows1, didx1, acc, ssem1, ns)
        return 0

    lax.fori_loop(0, N_PAIRS, pair, 0)
    _drain_scatters(rows1, didx1, acc, ssem1, ns)


_AGG1_SCRATCH = [
    pltpu.VMEM((SPC_E, 128), jnp.int32),
    pltpu.VMEM((SPC_E, 128), jnp.int32),
    pltpu.VMEM((SPC_E, 128), jnp.int32),
    pltpu.VMEM((SPC_E, 128), jnp.int32),
    pltpu.VMEM((SPC_E * 128, 8), jnp.float32),
    pltpu.VMEM((SPC_E * 128, 8), jnp.float32),
    pltpu.VMEM_SHARED((FACC_ROWS, 8), jnp.float32),
    pltpu.SemaphoreType.DMA,
    pltpu.SemaphoreType.DMA,
    pltpu.SemaphoreType.DMA,
    pltpu.SemaphoreType.DMA,
]


@functools.partial(
    pl.kernel,
    out_type=jax.ShapeDtypeStruct((2 * FACC_ROWS, 8), jnp.float32),
    mesh=_MESH, scratch_types=_AGG1_SCRATCH, compiler_params=_SC_PARAMS)
def _sc_agg1(src_hbm, dst_hbm, table_hbm, zeros_hbm, out,
             sidx0, sidx1, didx0, didx1, rows0, rows1, acc,
             gsem0, gsem1, ssem0, ssem1):
    c = lax.axis_index("c")
    s = lax.axis_index("s")
    pltpu.sync_copy(zeros_hbm, acc.at[pl.ds(s * FSPAN, FSPAN)])
    plsc.subcore_barrier()
    tile_base = (c * NS + s) * TSTRIDE_E
    _agg_pipeline(table_hbm, src_hbm, 0, dst_hbm, 0, tile_base,
                  sidx0, sidx1, didx0, didx1, rows0, rows1, acc,
                  gsem0, gsem1, ssem0, ssem1, SPC_E)
    plsc.subcore_barrier()
    ob = s * FSPAN
    pltpu.sync_copy(acc.at[pl.ds(ob, FSPAN)],
                    out.at[pl.ds(c * FACC_ROWS + ob, FSPAN)])


_AGG2_SCRATCH = [
    pltpu.VMEM((SPC_A, 128), jnp.int32),
    pltpu.VMEM((SPC_A, 128), jnp.int32),
    pltpu.VMEM((SPC_A, 128), jnp.int32),
    pltpu.VMEM((SPC_A, 128), jnp.int32),
    pltpu.VMEM((SPC_A * 128, D), jnp.float32),
    pltpu.VMEM((SPC_A * 128, D), jnp.float32),
    pltpu.VMEM_SHARED((ACC_ROWS, D), jnp.float32),
    pltpu.SemaphoreType.DMA,
    pltpu.SemaphoreType.DMA,
    pltpu.SemaphoreType.DMA,
    pltpu.SemaphoreType.DMA,
]


@functools.partial(
    pl.kernel,
    out_type=jax.ShapeDtypeStruct((4 * ACC_ROWS, D), jnp.float32),
    mesh=_MESH, scratch_types=_AGG2_SCRATCH, compiler_params=_SC_PARAMS)
def _sc_agg2(s2cat_hbm, lcat_hbm, table_hbm, zeros_hbm, out,
             sidx0, sidx1, didx0, didx1, rows0, rows1, acc,
             gsem0, gsem1, ssem0, ssem1):
    c = lax.axis_index("c")
    s = lax.axis_index("s")
    tile_base = s * TSTRIDE_A
    for k in range(2):
        pltpu.sync_copy(zeros_hbm, acc.at[pl.ds(s * SPAN, SPAN)])
        plsc.subcore_barrier()
        _agg_pipeline(table_hbm, s2cat_hbm, c * ER, lcat_hbm, k * ER,
                      tile_base, sidx0, sidx1, didx0, didx1, rows0, rows1,
                      acc, gsem0, gsem1, ssem0, ssem1, SPC_A)
        plsc.subcore_barrier()
        ob = s * SPAN
        pltpu.sync_copy(
            acc.at[pl.ds(ob, SPAN)],
            out.at[pl.ds((2 * c + k) * ACC_ROWS + ob, SPAN)])


_BLK = 5000
_GRID = NN // _BLK
_IBLK = 1568
_IGRID = ER // _IBLK


def _elu(t):
    return jnp.where(t > 0, t, jnp.exp(jnp.minimum(t, 0.0)) - 1.0)


def _row_spec(w, blk=_BLK):
    return pl.BlockSpec((blk, w), lambda i: (i, 0))


def _full_spec(shape):
    return pl.BlockSpec(shape, lambda i: tuple(0 for _ in shape))


def _tc_idx_body(src, dst, s2_o, l_o):
    k = pl.program_id(0)
    sv = src[:, :]
    dv = dst[:, :]
    s2_o[:, :] = sv + k * FACC_ROWS
    lo = dv - k * HALF
    ok = (lo >= 0) & (lo < HALF)
    dummy = HALF + (dv & 31)
    l_o[:, :] = jnp.where(ok, lo, dummy)


def _tc_a_body(d0, d1, x, dis_o, xs_o):
    deg = d0[:, :] + d1[:, :] + 1.0
    dis = lax.rsqrt(deg)
    dis_o[:, :] = dis
    xs_o[:, :] = x[:, :] * dis


def _tc_b_body(ag0, ag1, xs, dis, w1, b1, w2, lo_o, hi_o):
    d = dis[:, :]
    a1 = d * (ag0[:, :] + ag1[:, :] + xs[:, :])
    h1 = _elu(jnp.dot(a1, w1[:, :], preferred_element_type=jnp.float32)
              + b1[:, :])
    g = jnp.dot(h1, w2[:, :], preferred_element_type=jnp.float32) * d
    lo_o[:, :] = g[:, :16]
    hi_o[:, :] = g[:, 16:]


def _tc_c_body(alo, alo_h, ahi, ahi_h, glo, glo_h, ghi, ghi_h, dis, dis_h,
               b2, w0, w1, w2, bc, wf, bf, out_o):
    def h2(al, ah, gl, gh, dd):
        agg = jnp.concatenate([al + gl, ah + gh], axis=1)
        return _elu(dd * agg + b2[:, :])

    main = h2(alo[:, :], ahi[:, :], glo[:, :], ghi[:, :], dis[:, :])
    halo = h2(alo_h[:, :], ahi_h[:, :], glo_h[:, :], ghi_h[:, :], dis_h[:, :])
    hcat = jnp.concatenate([main, halo], axis=0)
    y = (jnp.dot(hcat[:_BLK], w0[:, :], preferred_element_type=jnp.float32)
         + jnp.dot(hcat[1:_BLK + 1], w1[:, :],
                   preferred_element_type=jnp.float32)
         + jnp.dot(hcat[2:_BLK + 2], w2[:, :],
                   preferred_element_type=jnp.float32)
         + bc[:, :])
    y = jnp.maximum(y, 0.0)
    out_o[:, :] = jnp.dot(y, wf[:, :], preferred_element_type=jnp.float32) \
        + bf[:, :]


def _halves(arr):
    return jnp.concatenate([arr[:HALF], arr[ACC_ROWS:ACC_ROWS + HALF],
                            jnp.zeros((FACC_ROWS - NN, D), jnp.float32)],
                           axis=0)


def _halo_specs(w):
    return [pl.BlockSpec((_BLK, w), lambda i: (i, 0)),
            pl.BlockSpec((8, w), lambda i: (_BLK // 8 * (i + 1), 0))]


def kernel(x, edge_index, W1, b1, W2, b2, Wc, bc, Wf, bf):
    src = edge_index[0]
    dst = edge_index[1]
    pad = E_PAD - EE
    src_p = jnp.concatenate(
        [src, jnp.zeros((pad,), jnp.int32)]).reshape(ER, 128)
    dst_p = jnp.concatenate(
        [dst, jnp.full((pad,), NN, jnp.int32)]).reshape(ER, 128)

    s2cat, lcat = pl.pallas_call(
        _tc_idx_body,
        grid=(2, _IGRID),
        in_specs=[pl.BlockSpec((_IBLK, 128), lambda k, i: (i, 0)),
                  pl.BlockSpec((_IBLK, 128), lambda k, i: (i, 0))],
        out_specs=[pl.BlockSpec((_IBLK, 128), lambda k, i: (k * _IGRID + i, 0)),
                   pl.BlockSpec((_IBLK, 128), lambda k, i: (k * _IGRID + i, 0))],
        out_shape=[jax.ShapeDtypeStruct((2 * ER, 128), jnp.int32),
                   jax.ShapeDtypeStruct((2 * ER, 128), jnp.int32)],
    )(src_p, dst_p)

    ones_e = jnp.ones((SPC_E * 128,), jnp.float32)
    zeros_1 = jnp.zeros((FSPAN,), jnp.float32)
    zeros_8 = jnp.zeros((FSPAN, 8), jnp.float32)
    zeros_16 = jnp.zeros((SPAN, D), jnp.float32)

    dgout = _sc_degree(dst_p, ones_e, zeros_1)
    d0 = dgout[:NN].reshape(NN, 1)
    d1 = dgout[FACC_ROWS:FACC_ROWS + NN].reshape(NN, 1)

    dis, xs = pl.pallas_call(
        _tc_a_body,
        grid=(_GRID,),
        in_specs=[_row_spec(1), _row_spec(1), _row_spec(8)],
        out_specs=[_row_spec(1), _row_spec(8)],
        out_shape=[jax.ShapeDtypeStruct((FACC_ROWS, 1), jnp.float32),
                   jax.ShapeDtypeStruct((NN, 8), jnp.float32)],
    )(d0, d1, x)

    agout = _sc_agg1(src_p, dst_p, xs, zeros_8)
    ag0 = agout[:NN]
    ag1 = agout[FACC_ROWS:FACC_ROWS + NN]

    glo, ghi = pl.pallas_call(
        _tc_b_body,
        grid=(_GRID,),
        in_specs=[_row_spec(8), _row_spec(8), _row_spec(8), _row_spec(1),
                  _full_spec((8, 64)), _full_spec((1, 64)),
                  _full_spec((64, 32))],
        out_specs=[_row_spec(D), _row_spec(D)],
        out_shape=[jax.ShapeDtypeStruct((FACC_ROWS, D), jnp.float32),
                   jax.ShapeDtypeStruct((FACC_ROWS, D), jnp.float32)],
    )(ag0, ag1, xs, dis, W1, b1.reshape(1, 64), W2)

    table2 = jnp.concatenate([glo, ghi], axis=0)
    a2 = _sc_agg2(s2cat, lcat, table2, zeros_16)
    alo = _halves(a2[: 2 * ACC_ROWS])
    ahi = _halves(a2[2 * ACC_ROWS:])

    out = pl.pallas_call(
        _tc_c_body,
        grid=(_GRID,),
        in_specs=(_halo_specs(D) + _halo_specs(D) + _halo_specs(D)
                  + _halo_specs(D) + _halo_specs(1)
                  + [_full_spec((1, 32)),
                     _full_spec((32, 16)), _full_spec((32, 16)),
                     _full_spec((32, 16)), _full_spec((1, 16)),
                     _full_spec((16, 22)), _full_spec((1, 22))]),
        out_specs=_row_spec(22),
        out_shape=jax.ShapeDtypeStruct((NN, 22), jnp.float32),
    )(alo, alo, ahi, ahi, glo, glo, ghi, ghi, dis, dis,
      b2.reshape(1, 32),
      Wc[:, :, 0].T, Wc[:, :, 1].T, Wc[:, :, 2].T, bc.reshape(1, 16),
      Wf.T, bf.reshape(1, 22))

    return out[: NN - 2]
